# baseline (speedup 1.0000x reference)
import jax
import jax.numpy as jnp
from jax import lax
from jax.experimental import pallas as pl
from jax.experimental.pallas import tpu as pltpu
from jax.experimental.pallas import tpu_sc as plsc

B = 16384
COLS = 20
NC = 2
NS = 16
LANES = 16
NW = NC * NS
CHUNK = B // NW
GROUPS = CHUNK // LANES


def _body(x_hbm, emb_hbm, out_hbm, xbuf, embbuf, obuf):
    cid = lax.axis_index("c")
    sid = lax.axis_index("s")
    wid = sid * NC + cid

    pltpu.sync_copy(x_hbm.at[pl.ds(wid * (CHUNK * COLS), CHUNK * COLS)], xbuf)
    pltpu.sync_copy(emb_hbm, embbuf)

    lane = lax.iota(jnp.int32, LANES)
    lane_cols = lane * COLS
    lane3 = lane * 3

    def group(i, carry):
        rb = i * (LANES * COLS)
        cols = [plsc.load_gather(xbuf, [lane_cols + (rb + j)]) for j in range(COLS)]
        obj = cols[15:20]
        for g in range(3):
            acc = None
            for j in range(5):
                idx = cols[5 * g + j].astype(jnp.int32)
                w = plsc.load_gather(embbuf, [idx])
                t = w * obj[j]
                acc = t if acc is None else acc + t
            plsc.store_scatter(obuf, [lane3 + (i * (LANES * 3) + g)], acc)
        return carry

    lax.fori_loop(0, GROUPS, group, 0)

    pltpu.sync_copy(obuf, out_hbm.at[pl.ds(wid * (CHUNK * 3), CHUNK * 3)])


@jax.jit
def kernel(x, level_embedding):
    x_flat = x.reshape(-1)
    emb = jnp.pad(level_embedding.reshape(-1), (0, 11))
    mesh = plsc.VectorSubcoreMesh(
        core_axis_name="c", subcore_axis_name="s",
        num_cores=NC, num_subcores=NS,
    )
    run = pl.kernel(
        _body,
        out_type=jax.ShapeDtypeStruct((B * 3,), jnp.float32),
        mesh=mesh,
        scratch_types=[
            pltpu.VMEM((CHUNK * COLS,), jnp.float32),
            pltpu.VMEM((LANES,), jnp.float32),
            pltpu.VMEM((CHUNK * 3,), jnp.float32),
        ],
        compiler_params=pltpu.CompilerParams(needs_layout_passes=False),
    )
    out_flat = run(x_flat, emb)
    return out_flat.reshape(B, 3)
